# baseline (speedup 1.0000x reference)
import jax
import jax.numpy as jnp
from jax.experimental import pallas as pl
from jax.experimental.pallas import tpu as pltpu

_BATCH = 128
_SEQ = 181
_NS = 8
_FEAT = 32
_KW = 5
_ALPHA = 0.3

_SB = 8
_NB = _SB * _NS


def _lrelu(x):
    return jnp.where(x > 0, x, _ALPHA * x)


def _conv(x, W, b):
    NB, L, Cin = x.shape
    Cout = W.shape[-1]
    z = jnp.zeros((NB, 2, Cin), x.dtype)
    xp = jnp.concatenate([z, x, z], axis=1)
    acc = None
    for k in range(_KW):
        xs = xp[:, k:k + L, :].reshape(NB * L, Cin)
        d = jax.lax.dot_general(xs, W[k], (((1,), (0,)), ((), ())),
                                preferred_element_type=jnp.float32)
        acc = d if acc is None else acc + d
    return acc.reshape(NB, L, Cout) + b


def _maxpool3(x):
    NB, L, C = x.shape
    L2 = (L // 3) * 3
    return x[:, :L2, :].reshape(NB, L2 // 3, 3, C).max(axis=2)


def _clayer(h, W, b):
    return _lrelu(_conv(h, W[...], b[...])).astype(jnp.bfloat16)


def _body(x_ref, W1, b1, W2, b2, W3a, b3a, W3b, b3b, W4a, b4a, W4b, b4b,
          W5, b5, Wd, bd, o_ref):
    h = x_ref[...]
    h = _clayer(h, W1, b1)
    h = _clayer(h, W2, b2)
    h = _maxpool3(h)
    h = _clayer(h, W3a, b3a)
    h = _clayer(h, W3b, b3b)
    h = _maxpool3(h)
    h = _clayer(h, W4a, b4a)
    h = _clayer(h, W4b, b4b)
    h = _maxpool3(h)
    h = _lrelu(_conv(h, W5[...], b5[...]))
    pooled = h.reshape(_SB, _NS * 6, 512).sum(axis=1) * (1.0 / (_NS * 6))
    logits = jax.lax.dot_general(pooled, Wd[...], (((1,), (0,)), ((), ())),
                                 preferred_element_type=jnp.float32) + bd[...]
    o_ref[...] = jax.nn.sigmoid(logits)


def _const_spec(shape):
    return pl.BlockSpec(shape, lambda i: (0,) * len(shape))


def kernel(sensor_features, sensor_indices, W1, b1, W2, b2, W3a, b3a,
           W3b, b3b, W4a, b4a, W4b, b4b, W5, b5, Wd, bd):
    x = sensor_features.reshape(_BATCH, _SEQ, _NS, _FEAT)
    x = x.transpose(0, 2, 1, 3).reshape(_BATCH * _NS, _SEQ, _FEAT)
    x = x.astype(jnp.bfloat16)

    biases = [b.reshape(1, -1) for b in (b1, b2, b3a, b3b, b4a, b4b, b5)]
    bdr = bd.reshape(1, 1)

    in_specs = [pl.BlockSpec((_NB, _SEQ, _FEAT), lambda i: (i, 0, 0))]
    weights = tuple(W.astype(jnp.bfloat16)
                    for W in (W1, W2, W3a, W3b, W4a, W4b, W5))
    ordered = []
    for W, b in zip(weights, biases):
        in_specs.append(_const_spec(W.shape))
        in_specs.append(_const_spec(b.shape))
        ordered.extend([W, b])
    in_specs.append(_const_spec(Wd.shape))
    in_specs.append(_const_spec(bdr.shape))
    ordered.extend([Wd, bdr])

    out = pl.pallas_call(
        _body,
        grid=(_BATCH // _SB,),
        in_specs=in_specs,
        out_specs=pl.BlockSpec((_SB, 1), lambda i: (i, 0)),
        out_shape=jax.ShapeDtypeStruct((_BATCH, 1), jnp.float32),
        compiler_params=pltpu.CompilerParams(
            dimension_semantics=("parallel",)),
    )(x, *ordered)
    return out


# speedup vs baseline: 2.4734x; 2.4734x over previous
import jax
import jax.numpy as jnp
from jax.experimental import pallas as pl
from jax.experimental.pallas import tpu as pltpu

_BATCH = 128
_SEQ = 181
_NS = 8
_FEAT = 32
_KW = 5
_ALPHA = 0.3

_SB = 8
_NB = _SB * _NS
_LP0 = 184


def _conv_im2col(x, Wc, b, L):
    NB, Lp, Cin = x.shape
    Cout = Wc.shape[-1]
    z = jnp.zeros((NB, 2, Cin), x.dtype)
    xp = jnp.concatenate([z, x, z], axis=1)
    cols = [xp[:, k:k + Lp, :] for k in range(_KW)]
    x5 = jnp.concatenate(cols, axis=-1)
    d = jax.lax.dot_general(x5.reshape(NB * Lp, _KW * Cin), Wc,
                            (((1,), (0,)), ((), ())),
                            preferred_element_type=jnp.float32)
    return d.reshape(NB, Lp, Cout) + b


def _clayer(h, Wc, b, L, mask):
    y = _conv_im2col(h, Wc[...], b[...], L)
    y = jnp.where(y > 0, y, _ALPHA * y).astype(jnp.bfloat16)
    if mask:
        Lp = y.shape[1]
        row = jax.lax.broadcasted_iota(jnp.int32, (1, Lp, 1), 1)
        y = jnp.where(row < L, y, jnp.bfloat16(0))
    return y


def _pool_pad(h, L, pad_to):
    NB, Lp, C = h.shape
    L2 = (L // 3) * 3
    p = h[:, :L2, :].reshape(NB, L2 // 3, 3, C).max(axis=2)
    z = jnp.zeros((NB, pad_to - L2 // 3, C), h.dtype)
    return jnp.concatenate([p, z], axis=1)


def _body(x_ref, W1, b1, W2, b2, W3a, b3a, W3b, b3b, W4a, b4a, W4b, b4b,
          W5, b5, Wd, bd, o_ref):
    h = x_ref[...]
    h = _clayer(h, W1, b1, _SEQ, mask=True)
    h = _clayer(h, W2, b2, _SEQ, mask=False)
    h = _pool_pad(h, _SEQ, 64)
    h = _clayer(h, W3a, b3a, 60, mask=True)
    h = _clayer(h, W3b, b3b, 60, mask=False)
    h = _pool_pad(h, 60, 24)
    h = _clayer(h, W4a, b4a, 20, mask=True)
    h = _clayer(h, W4b, b4b, 20, mask=False)
    h = _pool_pad(h, 20, 8)
    y = _conv_im2col(h, W5[...], b5[...], 6)
    y = jnp.where(y > 0, y, _ALPHA * y)
    s = y[:, 0:6, :].sum(axis=1)
    pooled = s.reshape(_SB, _NS, 512).sum(axis=1) * (1.0 / (_NS * 6))
    logits = jax.lax.dot_general(pooled, Wd[...], (((1,), (0,)), ((), ())),
                                 preferred_element_type=jnp.float32) + bd[...]
    o_ref[...] = jax.nn.sigmoid(logits)


def _const_spec(shape):
    return pl.BlockSpec(shape, lambda i: (0,) * len(shape))


def kernel(sensor_features, sensor_indices, W1, b1, W2, b2, W3a, b3a,
           W3b, b3b, W4a, b4a, W4b, b4b, W5, b5, Wd, bd):
    x = sensor_features.reshape(_BATCH, _SEQ, _NS, _FEAT)
    x = x.transpose(0, 2, 1, 3).reshape(_BATCH * _NS, _SEQ, _FEAT)
    x = jnp.pad(x, ((0, 0), (0, _LP0 - _SEQ), (0, 0))).astype(jnp.bfloat16)

    biases = [b.reshape(1, -1) for b in (b1, b2, b3a, b3b, b4a, b4b, b5)]
    bdr = bd.reshape(1, 1)
    weights = tuple(W.astype(jnp.bfloat16).reshape(-1, W.shape[-1])
                    for W in (W1, W2, W3a, W3b, W4a, W4b, W5))

    in_specs = [pl.BlockSpec((_NB, _LP0, _FEAT), lambda i: (i, 0, 0))]
    ordered = []
    for W, b in zip(weights, biases):
        in_specs.append(_const_spec(W.shape))
        in_specs.append(_const_spec(b.shape))
        ordered.extend([W, b])
    in_specs.append(_const_spec(Wd.shape))
    in_specs.append(_const_spec(bdr.shape))
    ordered.extend([Wd, bdr])

    out = pl.pallas_call(
        _body,
        grid=(_BATCH // _SB,),
        in_specs=in_specs,
        out_specs=pl.BlockSpec((_SB, 1), lambda i: (i, 0)),
        out_shape=jax.ShapeDtypeStruct((_BATCH, 1), jnp.float32),
        compiler_params=pltpu.CompilerParams(
            dimension_semantics=("parallel",)),
    )(x, *ordered)
    return out


# speedup vs baseline: 6.9358x; 2.8041x over previous
import jax
import jax.numpy as jnp
from jax.experimental import pallas as pl
from jax.experimental.pallas import tpu as pltpu

_BATCH = 128
_SEQ = 181
_NS = 8
_FEAT = 32
_KW = 5
_ALPHA = 0.3

_SB = 8
_ROWS0 = _SEQ * _NS


def _conv_lrelu(x, Wc, b):
    SB, R, Cin = x.shape
    z = jnp.zeros((SB, 2 * _NS, Cin), x.dtype)
    xp = jnp.concatenate([z, x, z], axis=1)
    cols = [xp[:, k * _NS:k * _NS + R, :] for k in range(_KW)]
    x5 = jnp.concatenate(cols, axis=-1)
    d = jax.lax.dot_general(x5.reshape(SB * R, _KW * Cin), Wc,
                            (((1,), (0,)), ((), ())),
                            preferred_element_type=jnp.float32)
    y = d + b
    y = jnp.where(y > 0, y, _ALPHA * y).astype(jnp.bfloat16)
    return y.reshape(SB, R, Wc.shape[-1])


def _maxpool3(h, T):
    SB, R, C = h.shape
    T2 = (T // 3) * 3
    g = h[:, :T2 * _NS, :].reshape(SB, T2 // 3, 3, _NS, C)
    return g.max(axis=2).reshape(SB, (T2 // 3) * _NS, C)


def _body(x_ref, W1, b1, W2, b2, W3a, b3a, W3b, b3b, W4a, b4a, W4b, b4b,
          W5, b5, Wd, bd, o_ref):
    h = x_ref[...].astype(jnp.bfloat16)
    h = _conv_lrelu(h, W1[...], b1[...])
    h = _conv_lrelu(h, W2[...], b2[...])
    h = _maxpool3(h, _SEQ)
    h = _conv_lrelu(h, W3a[...], b3a[...])
    h = _conv_lrelu(h, W3b[...], b3b[...])
    h = _maxpool3(h, 60)
    h = _conv_lrelu(h, W4a[...], b4a[...])
    h = _conv_lrelu(h, W4b[...], b4b[...])
    h = _maxpool3(h, 20)
    h = _conv_lrelu(h, W5[...], b5[...])
    pooled = h.astype(jnp.float32).sum(axis=1) * (1.0 / (6 * _NS))
    logits = jax.lax.dot_general(pooled, Wd[...], (((1,), (0,)), ((), ())),
                                 preferred_element_type=jnp.float32) + bd[...]
    o_ref[...] = jax.nn.sigmoid(logits)


def _const_spec(shape):
    return pl.BlockSpec(shape, lambda i: (0,) * len(shape))


def kernel(sensor_features, sensor_indices, W1, b1, W2, b2, W3a, b3a,
           W3b, b3b, W4a, b4a, W4b, b4b, W5, b5, Wd, bd):
    x = sensor_features.reshape(_BATCH, _ROWS0, _FEAT)

    biases = [b.reshape(1, -1) for b in (b1, b2, b3a, b3b, b4a, b4b, b5)]
    bdr = bd.reshape(1, 1)
    weights = tuple(W.astype(jnp.bfloat16).reshape(-1, W.shape[-1])
                    for W in (W1, W2, W3a, W3b, W4a, W4b, W5))

    in_specs = [pl.BlockSpec((_SB, _ROWS0, _FEAT), lambda i: (i, 0, 0))]
    ordered = []
    for W, b in zip(weights, biases):
        in_specs.append(_const_spec(W.shape))
        in_specs.append(_const_spec(b.shape))
        ordered.extend([W, b])
    in_specs.append(_const_spec(Wd.shape))
    in_specs.append(_const_spec(bdr.shape))
    ordered.extend([Wd, bdr])

    out = pl.pallas_call(
        _body,
        grid=(_BATCH // _SB,),
        in_specs=in_specs,
        out_specs=pl.BlockSpec((_SB, 1), lambda i: (i, 0)),
        out_shape=jax.ShapeDtypeStruct((_BATCH, 1), jnp.float32),
        compiler_params=pltpu.CompilerParams(
            dimension_semantics=("parallel",)),
    )(x, *ordered)
    return out


# speedup vs baseline: 6.9728x; 1.0053x over previous
import jax
import jax.numpy as jnp
from jax.experimental import pallas as pl
from jax.experimental.pallas import tpu as pltpu

_BATCH = 128
_SEQ = 181
_NS = 8
_FEAT = 32
_KW = 5
_ALPHA = 0.3

_SB = 16
_ROWS0 = _SEQ * _NS


def _conv_lrelu(x, Wc, b):
    SB, R, Cin = x.shape
    z = jnp.zeros((SB, 2 * _NS, Cin), x.dtype)
    xp = jnp.concatenate([z, x, z], axis=1)
    cols = [xp[:, k * _NS:k * _NS + R, :] for k in range(_KW)]
    x5 = jnp.concatenate(cols, axis=-1)
    d = jax.lax.dot_general(x5.reshape(SB * R, _KW * Cin), Wc,
                            (((1,), (0,)), ((), ())),
                            preferred_element_type=jnp.float32)
    y = d + b
    y = jnp.where(y > 0, y, _ALPHA * y).astype(jnp.bfloat16)
    return y.reshape(SB, R, Wc.shape[-1])


def _maxpool3(h, T):
    SB, R, C = h.shape
    T2 = (T // 3) * 3
    g = h[:, :T2 * _NS, :].reshape(SB, T2 // 3, 3, _NS, C)
    return g.max(axis=2).reshape(SB, (T2 // 3) * _NS, C)


def _body(x_ref, W1, b1, W2, b2, W3a, b3a, W3b, b3b, W4a, b4a, W4b, b4b,
          W5, b5, Wd, bd, o_ref):
    h = x_ref[...].astype(jnp.bfloat16)
    h = _conv_lrelu(h, W1[...], b1[...])
    h = _conv_lrelu(h, W2[...], b2[...])
    h = _maxpool3(h, _SEQ)
    h = _conv_lrelu(h, W3a[...], b3a[...])
    h = _conv_lrelu(h, W3b[...], b3b[...])
    h = _maxpool3(h, 60)
    h = _conv_lrelu(h, W4a[...], b4a[...])
    h = _conv_lrelu(h, W4b[...], b4b[...])
    h = _maxpool3(h, 20)
    h = _conv_lrelu(h, W5[...], b5[...])
    pooled = h.astype(jnp.float32).sum(axis=1) * (1.0 / (6 * _NS))
    logits = jax.lax.dot_general(pooled, Wd[...], (((1,), (0,)), ((), ())),
                                 preferred_element_type=jnp.float32) + bd[...]
    o_ref[...] = jax.nn.sigmoid(logits)


def _const_spec(shape):
    return pl.BlockSpec(shape, lambda i: (0,) * len(shape))


def kernel(sensor_features, sensor_indices, W1, b1, W2, b2, W3a, b3a,
           W3b, b3b, W4a, b4a, W4b, b4b, W5, b5, Wd, bd):
    x = sensor_features.reshape(_BATCH, _ROWS0, _FEAT)

    biases = [b.reshape(1, -1) for b in (b1, b2, b3a, b3b, b4a, b4b, b5)]
    bdr = bd.reshape(1, 1)
    weights = tuple(W.astype(jnp.bfloat16).reshape(-1, W.shape[-1])
                    for W in (W1, W2, W3a, W3b, W4a, W4b, W5))

    in_specs = [pl.BlockSpec((_SB, _ROWS0, _FEAT), lambda i: (i, 0, 0))]
    ordered = []
    for W, b in zip(weights, biases):
        in_specs.append(_const_spec(W.shape))
        in_specs.append(_const_spec(b.shape))
        ordered.extend([W, b])
    in_specs.append(_const_spec(Wd.shape))
    in_specs.append(_const_spec(bdr.shape))
    ordered.extend([Wd, bdr])

    out = pl.pallas_call(
        _body,
        grid=(_BATCH // _SB,),
        in_specs=in_specs,
        out_specs=pl.BlockSpec((_SB, 1), lambda i: (i, 0)),
        out_shape=jax.ShapeDtypeStruct((_BATCH, 1), jnp.float32),
        compiler_params=pltpu.CompilerParams(
            dimension_semantics=("parallel",)),
    )(x, *ordered)
    return out


# speedup vs baseline: 6.9732x; 1.0001x over previous
import jax
import jax.numpy as jnp
from jax.experimental import pallas as pl
from jax.experimental.pallas import tpu as pltpu

_BATCH = 128
_SEQ = 181
_NS = 8
_FEAT = 32
_KW = 5
_ALPHA = 0.3

_SB = 16
_ROWS0 = _SEQ * _NS


def _conv_lrelu(x, Wc, b):
    SB, R, Cin = x.shape
    z = jnp.zeros((SB, 2 * _NS, Cin), x.dtype)
    xp = jnp.concatenate([z, x, z], axis=1)
    cols = [xp[:, k * _NS:k * _NS + R, :] for k in range(_KW)]
    x5 = jnp.concatenate(cols, axis=-1)
    d = jax.lax.dot_general(x5.reshape(SB * R, _KW * Cin), Wc,
                            (((1,), (0,)), ((), ())),
                            preferred_element_type=jnp.float32)
    y = d + b
    y = jnp.where(y > 0, y, _ALPHA * y).astype(jnp.bfloat16)
    return y.reshape(SB, R, Wc.shape[-1])


def _maxpool3(h, T):
    SB, R, C = h.shape
    T2 = (T // 3) * 3
    g = h[:, :T2 * _NS, :].reshape(SB, T2 // 3, 3, _NS, C)
    return g.max(axis=2).reshape(SB, (T2 // 3) * _NS, C)


def _body(x_ref, W1, b1, W2, b2, W3a, b3a, W3b, b3b, W4a, b4a, W4b, b4b,
          W5, b5, Wd, bd, o_ref):
    h = x_ref[...].astype(jnp.bfloat16)
    h = _conv_lrelu(h, W1[...], b1[...])
    h = _conv_lrelu(h, W2[...], b2[...])
    h = _maxpool3(h, _SEQ)
    h = _conv_lrelu(h, W3a[...], b3a[...])
    h = _conv_lrelu(h, W3b[...], b3b[...])
    h = _maxpool3(h, 60)
    h = _conv_lrelu(h, W4a[...], b4a[...])
    h = _conv_lrelu(h, W4b[...], b4b[...])
    h = _maxpool3(h, 20)
    h = _conv_lrelu(h, W5[...], b5[...])
    pooled = h.astype(jnp.float32).sum(axis=1) * (1.0 / (6 * _NS))
    logits = jax.lax.dot_general(pooled, Wd[...], (((1,), (0,)), ((), ())),
                                 preferred_element_type=jnp.float32) + bd[...]
    o_ref[...] = jax.nn.sigmoid(logits)


def _const_spec(shape):
    return pl.BlockSpec(shape, lambda i: (0,) * len(shape))


def kernel(sensor_features, sensor_indices, W1, b1, W2, b2, W3a, b3a,
           W3b, b3b, W4a, b4a, W4b, b4b, W5, b5, Wd, bd):
    x = sensor_features.reshape(_BATCH, _ROWS0, _FEAT)

    biases = [b.reshape(1, -1) for b in (b1, b2, b3a, b3b, b4a, b4b, b5)]
    bdr = bd.reshape(1, 1)
    weights = tuple(W.astype(jnp.bfloat16).reshape(-1, W.shape[-1])
                    for W in (W1, W2, W3a, W3b, W4a, W4b, W5))

    in_specs = [pl.BlockSpec((_SB, _ROWS0, _FEAT), lambda i: (i, 0, 0))]
    ordered = []
    for W, b in zip(weights, biases):
        in_specs.append(_const_spec(W.shape))
        in_specs.append(_const_spec(b.shape))
        ordered.extend([W, b])
    in_specs.append(_const_spec(Wd.shape))
    in_specs.append(_const_spec(bdr.shape))
    ordered.extend([Wd, bdr])

    out = pl.pallas_call(
        _body,
        grid=(_BATCH // _SB,),
        in_specs=in_specs,
        out_specs=pl.BlockSpec((_SB, 1), lambda i: (i, 0)),
        out_shape=jax.ShapeDtypeStruct((_BATCH, 1), jnp.float32),
        compiler_params=pltpu.CompilerParams(
            dimension_semantics=("arbitrary",)),
    )(x, *ordered)
    return out


# speedup vs baseline: 8.4343x; 1.2095x over previous
import jax
import jax.numpy as jnp
from jax.experimental import pallas as pl
from jax.experimental.pallas import tpu as pltpu

_BATCH = 128
_SEQ = 181
_NS = 8
_FEAT = 32
_KW = 5
_ALPHA = 0.3

_SB = 16
_ROWS0 = _SEQ * _NS


def _lrelu_bf16(y):
    return jnp.where(y > 0, y, _ALPHA * y).astype(jnp.bfloat16)


def _conv_folded(x, Wp, bp, F, Cin):
    SB, R, L = x.shape
    z = jnp.zeros((SB, _NS, L), x.dtype)
    xp = jnp.concatenate([z, x, z], axis=1)
    left = xp[:, 0:R, (F - 2) * Cin:]
    mid = xp[:, _NS:_NS + R, :]
    right = xp[:, 2 * _NS:2 * _NS + R, 0:2 * Cin]
    xi = jnp.concatenate([left, mid, right], axis=-1)
    d = jax.lax.dot_general(xi.reshape(SB * R, (F + 4) * Cin), Wp,
                            (((1,), (0,)), ((), ())),
                            preferred_element_type=jnp.float32)
    y = _lrelu_bf16(d + bp)
    return y.reshape(SB, R, Wp.shape[-1])


def _conv_lrelu(x, Wc, b):
    SB, R, Cin = x.shape
    z = jnp.zeros((SB, 2 * _NS, Cin), x.dtype)
    xp = jnp.concatenate([z, x, z], axis=1)
    cols = [xp[:, k * _NS:k * _NS + R, :] for k in range(_KW)]
    x5 = jnp.concatenate(cols, axis=-1)
    d = jax.lax.dot_general(x5.reshape(SB * R, _KW * Cin), Wc,
                            (((1,), (0,)), ((), ())),
                            preferred_element_type=jnp.float32)
    return _lrelu_bf16(d + b).reshape(SB, R, Wc.shape[-1])


def _maxpool3(h, T):
    SB, R, C = h.shape
    T2 = (T // 3) * 3
    g = h[:, :T2 * _NS, :].reshape(SB, T2 // 3, 3, _NS, C)
    return g.max(axis=2).reshape(SB, (T2 // 3) * _NS, C)


def _pool_a(h):
    SB, R, L = h.shape
    g = h[:, :45 * _NS, :].reshape(SB, 15, 3, _NS, L)
    A, B, C = g[:, :, 0], g[:, :, 1], g[:, :, 2]
    t1 = jnp.concatenate([A[..., 0:64], A[..., 192:256],
                          B[..., 128:192], C[..., 64:128]], axis=-1)
    t2 = jnp.concatenate([A[..., 64:128], B[..., 0:64],
                          B[..., 192:256], C[..., 128:192]], axis=-1)
    t3 = jnp.concatenate([A[..., 128:192], B[..., 64:128],
                          C[..., 0:64], C[..., 192:256]], axis=-1)
    return jnp.maximum(t1, jnp.maximum(t2, t3)).reshape(SB, 15 * _NS, L)


def _pool_b(h):
    SB, R, L = h.shape
    g = h.reshape(SB, 10, 3, _NS, L)
    A, B, C = g[:, :, 0], g[:, :, 1], g[:, :, 2]
    t1 = jnp.concatenate([A[..., 0:128], B[..., 128:256]], axis=-1)
    t2 = jnp.concatenate([A[..., 128:256], C[..., 0:128]], axis=-1)
    t3 = jnp.concatenate([B[..., 0:128], C[..., 128:256]], axis=-1)
    return jnp.maximum(t1, jnp.maximum(t2, t3)).reshape(SB, 10 * _NS, L)


def _halve_fold(h):
    SB, R, L = h.shape
    g = h.reshape(SB, R // _NS, 1, _NS, L)
    lo = g[..., 0:L // 2]
    hi = g[..., L // 2:L]
    return jnp.concatenate([lo, hi], axis=2).reshape(SB, 2 * R, L // 2)


def _body(x_ref, W1, b1, W2, b2, W3a, b3a, W3b, b3b, W4a, b4a, W4b, b4b,
          W5, b5, Wd, bd, o_ref):
    x = x_ref[...].astype(jnp.bfloat16)
    SB = x.shape[0]
    xz = jnp.concatenate(
        [x, jnp.zeros((SB, 3 * _NS, _FEAT), x.dtype)], axis=1)
    xr = xz.reshape(SB, 46, 4, _NS, _FEAT)
    h = jnp.concatenate([xr[:, :, j] for j in range(4)],
                        axis=-1).reshape(SB, 46 * _NS, 4 * _FEAT)
    h = _conv_folded(h, W1[...], b1[...], 4, _FEAT)
    lane = jax.lax.broadcasted_iota(jnp.int32, (1, 1, 256), 2)
    tail = jnp.where(lane < 64, h[:, 45 * _NS:46 * _NS, :], jnp.bfloat16(0))
    h = jnp.concatenate([h[:, 0:45 * _NS, :], tail], axis=1)
    h = _conv_folded(h, W2[...], b2[...], 4, 64)
    h = _pool_a(h)
    h = _halve_fold(h)
    h = _conv_folded(h, W3a[...], b3a[...], 2, 64)
    h = _conv_folded(h, W3b[...], b3b[...], 2, 128)
    h = _pool_b(h)
    h = _halve_fold(h)
    h = _conv_lrelu(h, W4a[...], b4a[...])
    h = _conv_lrelu(h, W4b[...], b4b[...])
    h = _maxpool3(h, 20)
    h = _conv_lrelu(h, W5[...], b5[...])
    pooled = h.astype(jnp.float32).sum(axis=1) * (1.0 / (6 * _NS))
    logits = jax.lax.dot_general(pooled, Wd[...], (((1,), (0,)), ((), ())),
                                 preferred_element_type=jnp.float32) + bd[...]
    o_ref[...] = jax.nn.sigmoid(logits)


def _pack_w_folded(W, F):
    _, Cin, Cout = W.shape
    Wb = W.astype(jnp.bfloat16)
    Wp = jnp.zeros(((F + 4) * Cin, F * Cout), jnp.bfloat16)
    for jo in range(F):
        for k in range(_KW):
            b = jo + k
            Wp = Wp.at[b * Cin:(b + 1) * Cin,
                       jo * Cout:(jo + 1) * Cout].set(Wb[k])
    return Wp


def _const_spec(shape):
    return pl.BlockSpec(shape, lambda i: (0,) * len(shape))


def kernel(sensor_features, sensor_indices, W1, b1, W2, b2, W3a, b3a,
           W3b, b3b, W4a, b4a, W4b, b4b, W5, b5, Wd, bd):
    x = sensor_features.reshape(_BATCH, _ROWS0, _FEAT)

    packed = [
        (_pack_w_folded(W1, 4), jnp.tile(b1.reshape(1, -1), (1, 4))),
        (_pack_w_folded(W2, 4), jnp.tile(b2.reshape(1, -1), (1, 4))),
        (_pack_w_folded(W3a, 2), jnp.tile(b3a.reshape(1, -1), (1, 2))),
        (_pack_w_folded(W3b, 2), jnp.tile(b3b.reshape(1, -1), (1, 2))),
    ]
    flat = [(W.astype(jnp.bfloat16).reshape(-1, W.shape[-1]),
             b.reshape(1, -1)) for W, b in
            ((W4a, b4a), (W4b, b4b), (W5, b5))]
    bdr = bd.reshape(1, 1)

    in_specs = [pl.BlockSpec((_SB, _ROWS0, _FEAT), lambda i: (i, 0, 0))]
    ordered = []
    for W, b in packed + flat:
        in_specs.append(_const_spec(W.shape))
        in_specs.append(_const_spec(b.shape))
        ordered.extend([W, b])
    in_specs.append(_const_spec(Wd.shape))
    in_specs.append(_const_spec(bdr.shape))
    ordered.extend([Wd, bdr])

    out = pl.pallas_call(
        _body,
        grid=(_BATCH // _SB,),
        in_specs=in_specs,
        out_specs=pl.BlockSpec((_SB, 1), lambda i: (i, 0)),
        out_shape=jax.ShapeDtypeStruct((_BATCH, 1), jnp.float32),
        compiler_params=pltpu.CompilerParams(
            dimension_semantics=("parallel",)),
    )(x, *ordered)
    return out


# speedup vs baseline: 8.4640x; 1.0035x over previous
import jax
import jax.numpy as jnp
from jax.experimental import pallas as pl
from jax.experimental.pallas import tpu as pltpu

_BATCH = 128
_SEQ = 181
_NS = 8
_FEAT = 32
_KW = 5
_ALPHA = 0.3

_SB = 16
_ROWS0 = _SEQ * _NS


def _lrelu_bf16(y):
    return jnp.maximum(y, _ALPHA * y).astype(jnp.bfloat16)


def _conv_folded(x, Wp, bp, F, Cin):
    SB, R, L = x.shape
    z = jnp.zeros((SB, _NS, L), x.dtype)
    xp = jnp.concatenate([z, x, z], axis=1)
    left = xp[:, 0:R, (F - 2) * Cin:]
    mid = xp[:, _NS:_NS + R, :]
    right = xp[:, 2 * _NS:2 * _NS + R, 0:2 * Cin]
    xi = jnp.concatenate([left, mid, right], axis=-1)
    d = jax.lax.dot_general(xi.reshape(SB * R, (F + 4) * Cin), Wp,
                            (((1,), (0,)), ((), ())),
                            preferred_element_type=jnp.float32)
    y = _lrelu_bf16(d + bp)
    return y.reshape(SB, R, Wp.shape[-1])


def _conv_lrelu(x, Wc, b):
    SB, R, Cin = x.shape
    z = jnp.zeros((SB, 2 * _NS, Cin), x.dtype)
    xp = jnp.concatenate([z, x, z], axis=1)
    cols = [xp[:, k * _NS:k * _NS + R, :] for k in range(_KW)]
    x5 = jnp.concatenate(cols, axis=-1)
    d = jax.lax.dot_general(x5.reshape(SB * R, _KW * Cin), Wc,
                            (((1,), (0,)), ((), ())),
                            preferred_element_type=jnp.float32)
    return _lrelu_bf16(d + b).reshape(SB, R, Wc.shape[-1])


def _maxpool3(h, T):
    SB, R, C = h.shape
    T2 = (T // 3) * 3
    g = h[:, :T2 * _NS, :].reshape(SB, T2 // 3, 3, _NS, C)
    return g.max(axis=2).reshape(SB, (T2 // 3) * _NS, C)


def _pool_a(h):
    SB, R, L = h.shape
    g = h[:, :45 * _NS, :].reshape(SB, 15, 3, _NS, L)
    A, B, C = g[:, :, 0], g[:, :, 1], g[:, :, 2]
    t1 = jnp.concatenate([A[..., 0:64], A[..., 192:256],
                          B[..., 128:192], C[..., 64:128]], axis=-1)
    t2 = jnp.concatenate([A[..., 64:128], B[..., 0:64],
                          B[..., 192:256], C[..., 128:192]], axis=-1)
    t3 = jnp.concatenate([A[..., 128:192], B[..., 64:128],
                          C[..., 0:64], C[..., 192:256]], axis=-1)
    return jnp.maximum(t1, jnp.maximum(t2, t3)).reshape(SB, 15 * _NS, L)


def _pool_b(h):
    SB, R, L = h.shape
    g = h.reshape(SB, 10, 3, _NS, L)
    A, B, C = g[:, :, 0], g[:, :, 1], g[:, :, 2]
    t1 = jnp.concatenate([A[..., 0:128], B[..., 128:256]], axis=-1)
    t2 = jnp.concatenate([A[..., 128:256], C[..., 0:128]], axis=-1)
    t3 = jnp.concatenate([B[..., 0:128], C[..., 128:256]], axis=-1)
    return jnp.maximum(t1, jnp.maximum(t2, t3)).reshape(SB, 10 * _NS, L)


def _halve_fold(h):
    SB, R, L = h.shape
    g = h.reshape(SB, R // _NS, 1, _NS, L)
    lo = g[..., 0:L // 2]
    hi = g[..., L // 2:L]
    return jnp.concatenate([lo, hi], axis=2).reshape(SB, 2 * R, L // 2)


def _body(x_ref, W1, b1, W2, b2, W3a, b3a, W3b, b3b, W4a, b4a, W4b, b4b,
          W5, b5, Wd, bd, o_ref):
    x = x_ref[...].astype(jnp.bfloat16)
    SB = x.shape[0]
    xz = jnp.concatenate(
        [x, jnp.zeros((SB, 3 * _NS, _FEAT), x.dtype)], axis=1)
    xr = xz.reshape(SB, 46, 4, _NS, _FEAT)
    h = jnp.concatenate([xr[:, :, j] for j in range(4)],
                        axis=-1).reshape(SB, 46 * _NS, 4 * _FEAT)
    h = _conv_folded(h, W1[...], b1[...], 4, _FEAT)
    lane = jax.lax.broadcasted_iota(jnp.int32, (1, 1, 256), 2)
    tail = jnp.where(lane < 64, h[:, 45 * _NS:46 * _NS, :], jnp.bfloat16(0))
    h = jnp.concatenate([h[:, 0:45 * _NS, :], tail], axis=1)
    h = _conv_folded(h, W2[...], b2[...], 4, 64)
    h = _pool_a(h)
    h = _halve_fold(h)
    h = _conv_folded(h, W3a[...], b3a[...], 2, 64)
    h = _conv_folded(h, W3b[...], b3b[...], 2, 128)
    h = _pool_b(h)
    h = _halve_fold(h)
    h = _conv_lrelu(h, W4a[...], b4a[...])
    h = _conv_lrelu(h, W4b[...], b4b[...])
    h = _maxpool3(h, 20)
    h = _conv_lrelu(h, W5[...], b5[...])
    pooled = h.astype(jnp.float32).sum(axis=1) * (1.0 / (6 * _NS))
    logits = jax.lax.dot_general(pooled, Wd[...], (((1,), (0,)), ((), ())),
                                 preferred_element_type=jnp.float32) + bd[...]
    o_ref[...] = jax.nn.sigmoid(logits)


def _pack_w_folded(W, F):
    _, Cin, Cout = W.shape
    Wb = W.astype(jnp.bfloat16)
    Wp = jnp.zeros(((F + 4) * Cin, F * Cout), jnp.bfloat16)
    for jo in range(F):
        for k in range(_KW):
            b = jo + k
            Wp = Wp.at[b * Cin:(b + 1) * Cin,
                       jo * Cout:(jo + 1) * Cout].set(Wb[k])
    return Wp


def _const_spec(shape):
    return pl.BlockSpec(shape, lambda i: (0,) * len(shape))


def kernel(sensor_features, sensor_indices, W1, b1, W2, b2, W3a, b3a,
           W3b, b3b, W4a, b4a, W4b, b4b, W5, b5, Wd, bd):
    x = sensor_features.reshape(_BATCH, _ROWS0, _FEAT)

    packed = [
        (_pack_w_folded(W1, 4), jnp.tile(b1.reshape(1, -1), (1, 4))),
        (_pack_w_folded(W2, 4), jnp.tile(b2.reshape(1, -1), (1, 4))),
        (_pack_w_folded(W3a, 2), jnp.tile(b3a.reshape(1, -1), (1, 2))),
        (_pack_w_folded(W3b, 2), jnp.tile(b3b.reshape(1, -1), (1, 2))),
    ]
    flat = [(W.astype(jnp.bfloat16).reshape(-1, W.shape[-1]),
             b.reshape(1, -1)) for W, b in
            ((W4a, b4a), (W4b, b4b), (W5, b5))]
    bdr = bd.reshape(1, 1)

    in_specs = [pl.BlockSpec((_SB, _ROWS0, _FEAT), lambda i: (i, 0, 0))]
    ordered = []
    for W, b in packed + flat:
        in_specs.append(_const_spec(W.shape))
        in_specs.append(_const_spec(b.shape))
        ordered.extend([W, b])
    in_specs.append(_const_spec(Wd.shape))
    in_specs.append(_const_spec(bdr.shape))
    ordered.extend([Wd, bdr])

    out = pl.pallas_call(
        _body,
        grid=(_BATCH // _SB,),
        in_specs=in_specs,
        out_specs=pl.BlockSpec((_SB, 1), lambda i: (i, 0)),
        out_shape=jax.ShapeDtypeStruct((_BATCH, 1), jnp.float32),
        compiler_params=pltpu.CompilerParams(
            dimension_semantics=("parallel",)),
    )(x, *ordered)
    return out
